# Initial kernel scaffold; baseline (speedup 1.0000x reference)
import jax
import jax.numpy as jnp
from jax.experimental import pallas as pl


def kernel(x, W, mean_features):
    raise NotImplementedError("write your pallas kernel here")


# speedup vs baseline: 1.0388x; 1.0388x over previous
import jax
import jax.numpy as jnp
from jax.experimental import pallas as pl
from jax.experimental.pallas import tpu as pltpu

_BLOCK_ROWS = 256


def _icarl_block(x_ref, w_ref, mt_ref, out_ref):
    x = x_ref[...]
    w = w_ref[...]
    mt = mt_ref[...]
    preds = jnp.dot(x, w, preferred_element_type=jnp.float32)
    pm = jnp.dot(preds, mt, preferred_element_type=jnp.float32)
    a2 = jnp.sum(preds * preds, axis=1, keepdims=True)
    b2 = jnp.sum(mt * mt, axis=0, keepdims=True)
    d2 = a2 + b2 - 2.0 * pm
    dist = jnp.sqrt(jnp.clip(d2, 0.0, None))
    min_d = jnp.min(dist, axis=1, keepdims=True)
    c = dist.shape[1]
    col = jax.lax.broadcasted_iota(jnp.int32, dist.shape, 1)
    idx = jnp.min(jnp.where(dist == min_d, col, c), axis=1, keepdims=True)
    out_ref[...] = (col == idx).astype(jnp.float32)


def kernel(x, W, mean_features):
    ns, d_in = x.shape
    nf = W.shape[1]
    c = mean_features.shape[0]
    mt = mean_features.T
    return pl.pallas_call(
        _icarl_block,
        grid=(ns // _BLOCK_ROWS,),
        in_specs=[
            pl.BlockSpec((_BLOCK_ROWS, d_in), lambda i: (i, 0)),
            pl.BlockSpec((d_in, nf), lambda i: (0, 0)),
            pl.BlockSpec((nf, c), lambda i: (0, 0)),
        ],
        out_specs=pl.BlockSpec((_BLOCK_ROWS, c), lambda i: (i, 0)),
        out_shape=jax.ShapeDtypeStruct((ns, c), jnp.float32),
        compiler_params=pltpu.CompilerParams(
            dimension_semantics=("parallel",)),
    )(x, W, mt)
